# baseline (speedup 1.0000x reference)
import functools

import jax
import jax.numpy as jnp
from jax import lax
from jax.experimental import pallas as pl
from jax.experimental.pallas import tpu as pltpu
from jax.experimental.pallas import tpu_sc as plsc

_N = 10000
_E = 320000
_D = 128
_NPAD = 10240
_CH = 80
_NTILES = 32
_CHUNKS = 125
_RPT = _NPAD // 16


def _segment_sum_sc(h, src1, dst1):
    mesh = plsc.VectorSubcoreMesh(core_axis_name="c", subcore_axis_name="s")

    @functools.partial(
        pl.kernel,
        out_type=jax.ShapeDtypeStruct((2, _NPAD, _D), jnp.float32),
        mesh=mesh,
        scratch_types=[
            pltpu.VMEM((_CH,), jnp.int32),
            pltpu.VMEM((_CH,), jnp.int32),
            pltpu.VMEM((_CH,), jnp.int32),
            pltpu.VMEM((_CH,), jnp.int32),
            pltpu.VMEM((_CH,), jnp.int32),
            pltpu.VMEM((_CH,), jnp.int32),
            pltpu.VMEM((_CH, _D), jnp.float32),
            pltpu.VMEM((_CH, _D), jnp.float32),
            pltpu.VMEM((16, _D), jnp.float32),
            pltpu.VMEM_SHARED((_NPAD, _D), jnp.float32),
            pltpu.SemaphoreType.DMA,
            pltpu.SemaphoreType.DMA,
            pltpu.SemaphoreType.DMA,
            pltpu.SemaphoreType.DMA,
            pltpu.SemaphoreType.DMA,
            pltpu.SemaphoreType.DMA,
        ],
    )
    def k(h_hbm, src_hbm, dst_hbm, out_hbm, sidx0, sidx1, didx0, didx1,
          didx2, didx3, rows0, rows1, zbuf, acc, semi0, semi1, semg0, semg1,
          sems0, sems1):
        cid = lax.axis_index("c")
        sid = lax.axis_index("s")
        wid = cid * 16 + sid
        ebase = wid * (_CHUNKS * _CH)

        def zstore(i, carry):
            zbuf[i // 8, pl.ds((i % 8) * 16, 16)] = jnp.zeros((16,), jnp.float32)
            return carry
        lax.fori_loop(0, 16 * (_D // 16), zstore, 0)

        def zcopy(j, carry):
            pltpu.sync_copy(zbuf, acc.at[pl.ds(sid * _RPT + j * 16, 16)])
            return carry
        lax.fori_loop(0, _RPT // 16, zcopy, 0)
        plsc.subcore_barrier()

        def fire_idx(c, sidx, didx, semi):
            off = ebase + c * _CH
            pltpu.async_copy(src_hbm.at[pl.ds(off, _CH)], sidx, semi)
            pltpu.async_copy(dst_hbm.at[pl.ds(off, _CH)], didx, semi)

        def wait_idx(sidx, didx, semi):
            pltpu.make_async_copy(src_hbm.at[pl.ds(0, _CH)], sidx, semi).wait()
            pltpu.make_async_copy(src_hbm.at[pl.ds(0, _CH)], didx, semi).wait()

        def fire_gather(sidx, rows, semg):
            return pltpu.async_copy(h_hbm.at[sidx], rows, semg)

        def wait_gather(sidx, rows, semg):
            pltpu.make_async_copy(h_hbm.at[sidx], rows, semg).wait()

        def fire_scatter(didx, rows, sems):
            return pltpu.async_copy(rows, acc.at[didx], sems, add=True)

        fire_idx(0, sidx0, didx0, semi0)
        fire_idx(1, sidx1, didx1, semi1)
        wait_idx(sidx0, didx0, semi0)
        fire_gather(sidx0, rows0, semg0)
        wait_idx(sidx1, didx1, semi1)
        fire_gather(sidx1, rows1, semg1)

        def body(kk, carry):
            c = kk * 4
            wait_gather(sidx0, rows0, semg0)
            d0 = fire_scatter(didx0, rows0, sems0)
            wait_gather(sidx1, rows1, semg1)
            d1 = fire_scatter(didx1, rows1, sems1)
            fire_idx(c + 2, sidx0, didx2, semi0)
            fire_idx(c + 3, sidx1, didx3, semi1)
            d0.wait()
            d1.wait()
            wait_idx(sidx0, didx2, semi0)
            fire_gather(sidx0, rows0, semg0)
            wait_idx(sidx1, didx3, semi1)
            fire_gather(sidx1, rows1, semg1)
            wait_gather(sidx0, rows0, semg0)
            d2 = fire_scatter(didx2, rows0, sems0)
            wait_gather(sidx1, rows1, semg1)
            d3 = fire_scatter(didx3, rows1, sems1)
            fire_idx(c + 4, sidx0, didx0, semi0)
            fire_idx(c + 5, sidx1, didx1, semi1)
            d2.wait()
            d3.wait()
            wait_idx(sidx0, didx0, semi0)
            fire_gather(sidx0, rows0, semg0)
            wait_idx(sidx1, didx1, semi1)
            fire_gather(sidx1, rows1, semg1)
            return carry
        lax.fori_loop(0, (_CHUNKS - 5) // 4, body, 0)

        ct = _CHUNKS - 5
        wait_gather(sidx0, rows0, semg0)
        d0 = fire_scatter(didx0, rows0, sems0)
        wait_gather(sidx1, rows1, semg1)
        d1 = fire_scatter(didx1, rows1, sems1)
        fire_idx(ct + 2, sidx0, didx2, semi0)
        fire_idx(ct + 3, sidx1, didx3, semi1)
        d0.wait()
        d1.wait()
        wait_idx(sidx0, didx2, semi0)
        fire_gather(sidx0, rows0, semg0)
        wait_idx(sidx1, didx3, semi1)
        fire_gather(sidx1, rows1, semg1)
        wait_gather(sidx0, rows0, semg0)
        d2 = fire_scatter(didx2, rows0, sems0)
        wait_gather(sidx1, rows1, semg1)
        d3 = fire_scatter(didx3, rows1, sems1)
        d2.wait()
        fire_idx(ct + 4, sidx0, didx0, semi0)
        wait_idx(sidx0, didx0, semi0)
        fire_gather(sidx0, rows0, semg0)
        wait_gather(sidx0, rows0, semg0)
        fire_scatter(didx0, rows0, sems0).wait()
        d3.wait()
        plsc.subcore_barrier()

        pltpu.sync_copy(acc.at[pl.ds(sid * _RPT, _RPT)],
                        out_hbm.at[cid, pl.ds(sid * _RPT, _RPT)])

    return k(h, src1, dst1)


def _dense_body(x_ref, agg_ref, w1_ref, b1_ref, al_ref, g_ref, be_ref,
                w2_ref, b2_ref, out_ref):
    h = x_ref[...] + agg_ref[0, :_N, :] + agg_ref[1, :_N, :]
    h = jnp.dot(h, w1_ref[...], preferred_element_type=jnp.float32) + b1_ref[...]
    mean = jnp.mean(h, axis=0, keepdims=True)
    cen = h - al_ref[...] * mean
    var = jnp.mean(cen * cen, axis=0, keepdims=True)
    h = g_ref[...] * cen / jnp.sqrt(var + 1e-5) + be_ref[...]
    h = jnp.maximum(h, 0.0)
    h = jnp.dot(h, w2_ref[...], preferred_element_type=jnp.float32) + b2_ref[...]
    out_ref[...] = jnp.maximum(h, 0.0)


def _dense_layer(x, agg, W1, b1, alpha, gamma, beta, W2, b2):
    return pl.pallas_call(
        _dense_body,
        out_shape=jax.ShapeDtypeStruct((_N, _D), jnp.float32),
    )(x, agg, W1, b1.reshape(1, _D), alpha.reshape(1, _D),
      gamma.reshape(1, _D), beta.reshape(1, _D), W2, b2.reshape(1, _D))


def kernel(x, edge_index, W1_0, b1_0, alpha_0, gamma_0, beta_0, W2_0, b2_0,
           W1_1, b1_1, alpha_1, gamma_1, beta_1, W2_1, b2_1):
    src1 = edge_index[0]
    dst1 = edge_index[1]
    agg0 = _segment_sum_sc(x, src1, dst1)
    h = _dense_layer(x, agg0, W1_0, b1_0, alpha_0, gamma_0, beta_0, W2_0, b2_0)
    agg1 = _segment_sum_sc(h, src1, dst1)
    h = _dense_layer(h, agg1, W1_1, b1_1, alpha_1, gamma_1, beta_1, W2_1, b2_1)
    return h


# speedup vs baseline: 1.0105x; 1.0105x over previous
import functools

import jax
import jax.numpy as jnp
from jax import lax
from jax.experimental import pallas as pl
from jax.experimental.pallas import tpu as pltpu
from jax.experimental.pallas import tpu_sc as plsc

_N = 10000
_E = 320000
_D = 128
_NPAD = 10240
_CH = 80
_NTILES = 32
_CHUNKS = 125
_RPT = _NPAD // 16


def _segment_sum_sc(h, src1, dst1):
    mesh = plsc.VectorSubcoreMesh(core_axis_name="c", subcore_axis_name="s")

    @functools.partial(
        pl.kernel,
        out_type=jax.ShapeDtypeStruct((2, _NPAD, _D), jnp.float32),
        mesh=mesh,
        scratch_types=[
            pltpu.VMEM((_CH,), jnp.int32),
            pltpu.VMEM((_CH,), jnp.int32),
            pltpu.VMEM((_CH,), jnp.int32),
            pltpu.VMEM((_CH,), jnp.int32),
            pltpu.VMEM((_CH,), jnp.int32),
            pltpu.VMEM((_CH,), jnp.int32),
            pltpu.VMEM((_CH, _D), jnp.float32),
            pltpu.VMEM((_CH, _D), jnp.float32),
            pltpu.VMEM((80, _D), jnp.float32),
            pltpu.VMEM_SHARED((_NPAD, _D), jnp.float32),
            pltpu.SemaphoreType.DMA,
            pltpu.SemaphoreType.DMA,
            pltpu.SemaphoreType.DMA,
            pltpu.SemaphoreType.DMA,
            pltpu.SemaphoreType.DMA,
            pltpu.SemaphoreType.DMA,
        ],
    )
    def k(h_hbm, src_hbm, dst_hbm, out_hbm, sidx0, sidx1, didx0, didx1,
          didx2, didx3, rows0, rows1, zbuf, acc, semi0, semi1, semg0, semg1,
          sems0, sems1):
        cid = lax.axis_index("c")
        sid = lax.axis_index("s")
        wid = cid * 16 + sid
        ebase = wid * (_CHUNKS * _CH)

        def zstore(i, carry):
            zbuf[i // 8, pl.ds((i % 8) * 16, 16)] = jnp.zeros((16,), jnp.float32)
            return carry
        lax.fori_loop(0, 80 * (_D // 16), zstore, 0)

        def fire_idx(c, sidx, didx, semi):
            off = ebase + c * _CH
            pltpu.async_copy(src_hbm.at[pl.ds(off, _CH)], sidx, semi)
            pltpu.async_copy(dst_hbm.at[pl.ds(off, _CH)], didx, semi)

        def wait_idx(sidx, didx, semi):
            pltpu.make_async_copy(src_hbm.at[pl.ds(0, _CH)], sidx, semi).wait()
            pltpu.make_async_copy(src_hbm.at[pl.ds(0, _CH)], didx, semi).wait()

        def fire_gather(sidx, rows, semg):
            return pltpu.async_copy(h_hbm.at[sidx], rows, semg)

        def wait_gather(sidx, rows, semg):
            pltpu.make_async_copy(h_hbm.at[sidx], rows, semg).wait()

        def fire_scatter(didx, rows, sems):
            return pltpu.async_copy(rows, acc.at[didx], sems, add=True)

        zd = [pltpu.async_copy(zbuf, acc.at[pl.ds(sid * _RPT + j * 80, 80)],
                               sems0) for j in range(_RPT // 80)]
        fire_idx(0, sidx0, didx0, semi0)
        fire_idx(1, sidx1, didx1, semi1)
        wait_idx(sidx0, didx0, semi0)
        fire_gather(sidx0, rows0, semg0)
        wait_idx(sidx1, didx1, semi1)
        fire_gather(sidx1, rows1, semg1)
        for d in zd:
            d.wait()
        plsc.subcore_barrier()

        def body(kk, carry):
            c = kk * 4
            wait_gather(sidx0, rows0, semg0)
            d0 = fire_scatter(didx0, rows0, sems0)
            wait_gather(sidx1, rows1, semg1)
            d1 = fire_scatter(didx1, rows1, sems1)
            fire_idx(c + 2, sidx0, didx2, semi0)
            fire_idx(c + 3, sidx1, didx3, semi1)
            d0.wait()
            d1.wait()
            wait_idx(sidx0, didx2, semi0)
            fire_gather(sidx0, rows0, semg0)
            wait_idx(sidx1, didx3, semi1)
            fire_gather(sidx1, rows1, semg1)
            wait_gather(sidx0, rows0, semg0)
            d2 = fire_scatter(didx2, rows0, sems0)
            wait_gather(sidx1, rows1, semg1)
            d3 = fire_scatter(didx3, rows1, sems1)
            fire_idx(c + 4, sidx0, didx0, semi0)
            fire_idx(c + 5, sidx1, didx1, semi1)
            d2.wait()
            d3.wait()
            wait_idx(sidx0, didx0, semi0)
            fire_gather(sidx0, rows0, semg0)
            wait_idx(sidx1, didx1, semi1)
            fire_gather(sidx1, rows1, semg1)
            return carry
        lax.fori_loop(0, (_CHUNKS - 5) // 4, body, 0)

        ct = _CHUNKS - 5
        wait_gather(sidx0, rows0, semg0)
        d0 = fire_scatter(didx0, rows0, sems0)
        wait_gather(sidx1, rows1, semg1)
        d1 = fire_scatter(didx1, rows1, sems1)
        fire_idx(ct + 2, sidx0, didx2, semi0)
        fire_idx(ct + 3, sidx1, didx3, semi1)
        d0.wait()
        d1.wait()
        wait_idx(sidx0, didx2, semi0)
        fire_gather(sidx0, rows0, semg0)
        wait_idx(sidx1, didx3, semi1)
        fire_gather(sidx1, rows1, semg1)
        wait_gather(sidx0, rows0, semg0)
        d2 = fire_scatter(didx2, rows0, sems0)
        wait_gather(sidx1, rows1, semg1)
        d3 = fire_scatter(didx3, rows1, sems1)
        d2.wait()
        fire_idx(ct + 4, sidx0, didx0, semi0)
        wait_idx(sidx0, didx0, semi0)
        fire_gather(sidx0, rows0, semg0)
        wait_gather(sidx0, rows0, semg0)
        fire_scatter(didx0, rows0, sems0).wait()
        d3.wait()
        plsc.subcore_barrier()

        pltpu.sync_copy(acc.at[pl.ds(sid * _RPT, _RPT)],
                        out_hbm.at[cid, pl.ds(sid * _RPT, _RPT)])

    return k(h, src1, dst1)


def _dense_body(x_ref, agg_ref, w1_ref, b1_ref, al_ref, g_ref, be_ref,
                w2_ref, b2_ref, out_ref):
    h = x_ref[...] + agg_ref[0, :_N, :] + agg_ref[1, :_N, :]
    h = jnp.dot(h, w1_ref[...], preferred_element_type=jnp.float32) + b1_ref[...]
    mean = jnp.mean(h, axis=0, keepdims=True)
    cen = h - al_ref[...] * mean
    var = jnp.mean(cen * cen, axis=0, keepdims=True)
    h = g_ref[...] * cen / jnp.sqrt(var + 1e-5) + be_ref[...]
    h = jnp.maximum(h, 0.0)
    h = jnp.dot(h, w2_ref[...], preferred_element_type=jnp.float32) + b2_ref[...]
    out_ref[...] = jnp.maximum(h, 0.0)


def _dense_layer(x, agg, W1, b1, alpha, gamma, beta, W2, b2):
    return pl.pallas_call(
        _dense_body,
        out_shape=jax.ShapeDtypeStruct((_N, _D), jnp.float32),
    )(x, agg, W1, b1.reshape(1, _D), alpha.reshape(1, _D),
      gamma.reshape(1, _D), beta.reshape(1, _D), W2, b2.reshape(1, _D))


def kernel(x, edge_index, W1_0, b1_0, alpha_0, gamma_0, beta_0, W2_0, b2_0,
           W1_1, b1_1, alpha_1, gamma_1, beta_1, W2_1, b2_1):
    src1 = edge_index[0]
    dst1 = edge_index[1]
    agg0 = _segment_sum_sc(x, src1, dst1)
    h = _dense_layer(x, agg0, W1_0, b1_0, alpha_0, gamma_0, beta_0, W2_0, b2_0)
    agg1 = _segment_sum_sc(h, src1, dst1)
    h = _dense_layer(h, agg1, W1_1, b1_1, alpha_1, gamma_1, beta_1, W2_1, b2_1)
    return h


# speedup vs baseline: 1.0879x; 1.0766x over previous
import functools

import jax
import jax.numpy as jnp
from jax import lax
from jax.experimental import pallas as pl
from jax.experimental.pallas import tpu as pltpu
from jax.experimental.pallas import tpu_sc as plsc

_N = 10000
_E = 320000
_D = 128
_NPAD = 10240
_CH = 80
_NTILES = 32
_CHUNKS = 125
_RPT = _NPAD // 16


def _segment_sum_sc(h, src1, dst1):
    mesh = plsc.VectorSubcoreMesh(core_axis_name="c", subcore_axis_name="s")

    @functools.partial(
        pl.kernel,
        out_type=jax.ShapeDtypeStruct((2, _NPAD, _D), jnp.float32),
        mesh=mesh,
        scratch_types=[
            pltpu.VMEM((3, _CH), jnp.int32),
            pltpu.VMEM((6, _CH), jnp.int32),
            pltpu.VMEM((3, _CH, _D), jnp.float32),
            pltpu.VMEM((16, _D), jnp.float32),
            pltpu.VMEM_SHARED((_NPAD, _D), jnp.float32),
            pltpu.SemaphoreType.DMA,
            pltpu.SemaphoreType.DMA,
            pltpu.SemaphoreType.DMA,
            pltpu.SemaphoreType.DMA,
            pltpu.SemaphoreType.DMA,
            pltpu.SemaphoreType.DMA,
            pltpu.SemaphoreType.DMA,
            pltpu.SemaphoreType.DMA,
            pltpu.SemaphoreType.DMA,
        ],
    )
    def k(h_hbm, src_hbm, dst_hbm, out_hbm, sidxb, didxb, rowsb, zbuf, acc,
          semi0, semi1, semi2, semg0, semg1, semg2, sems0, sems1, sems2):
        cid = lax.axis_index("c")
        sid = lax.axis_index("s")
        wid = cid * 16 + sid
        ebase = wid * (_CHUNKS * _CH)

        def zstore(i, carry):
            zbuf[i // 8, pl.ds((i % 8) * 16, 16)] = jnp.zeros((16,), jnp.float32)
            return carry
        lax.fori_loop(0, 16 * (_D // 16), zstore, 0)

        sidx = [sidxb.at[j] for j in range(3)]
        didx = [didxb.at[j] for j in range(6)]
        rows = [rowsb.at[j] for j in range(3)]
        semi = [semi0, semi1, semi2]
        semg = [semg0, semg1, semg2]
        sems = [sems0, sems1, sems2]

        def fire_idx(c, b, db):
            off = ebase + c * _CH
            pltpu.async_copy(src_hbm.at[pl.ds(off, _CH)], sidx[b], semi[b])
            pltpu.async_copy(dst_hbm.at[pl.ds(off, _CH)], didx[db], semi[b])

        def wait_idx(b, db):
            pltpu.make_async_copy(src_hbm.at[pl.ds(0, _CH)], sidx[b],
                                  semi[b]).wait()
            pltpu.make_async_copy(src_hbm.at[pl.ds(0, _CH)], didx[db],
                                  semi[b]).wait()

        def fire_gather(b):
            return pltpu.async_copy(h_hbm.at[sidx[b]], rows[b], semg[b])

        def wait_gather(b):
            pltpu.make_async_copy(h_hbm.at[sidx[b]], rows[b], semg[b]).wait()

        def fire_scatter(b, db):
            return pltpu.async_copy(rows[b], acc.at[didx[db]], sems[b],
                                    add=True)

        zd = [pltpu.async_copy(zbuf, acc.at[pl.ds(sid * _RPT + j * 16, 16)],
                               sems0) for j in range(_RPT // 16)]
        for b in range(3):
            fire_idx(b, b, b)
        for b in range(3):
            wait_idx(b, b)
            fire_gather(b)
        for d in zd:
            d.wait()
        plsc.subcore_barrier()

        def trio(c, dbs, nxt_c, nxt_dbs, fire_next):
            ds_ = []
            for b in range(3):
                wait_gather(b)
                ds_.append(fire_scatter(b, dbs[b]))
            if fire_next:
                for b in range(3):
                    fire_idx(nxt_c + b, b, nxt_dbs[b])
            for d in ds_:
                d.wait()
            if fire_next:
                for b in range(3):
                    wait_idx(b, nxt_dbs[b])
                    fire_gather(b)

        def body(kk, carry):
            c = kk * 6
            trio(c, (0, 1, 2), c + 3, (3, 4, 5), True)
            trio(c + 3, (3, 4, 5), c + 6, (0, 1, 2), True)
            return carry
        lax.fori_loop(0, (_CHUNKS - 5) // 6, body, 0)

        ct = _CHUNKS - 5
        ds_ = []
        for b in range(3):
            wait_gather(b)
            ds_.append(fire_scatter(b, b))
        for b in range(2):
            fire_idx(ct + 3 + b, b, 3 + b)
        for d in ds_:
            d.wait()
        for b in range(2):
            wait_idx(b, 3 + b)
            fire_gather(b)
        ds_ = []
        for b in range(2):
            wait_gather(b)
            ds_.append(fire_scatter(b, 3 + b))
        for d in ds_:
            d.wait()
        plsc.subcore_barrier()

        pltpu.sync_copy(acc.at[pl.ds(sid * _RPT, _RPT)],
                        out_hbm.at[cid, pl.ds(sid * _RPT, _RPT)])

    return k(h, src1, dst1)


def _dense_body(x_ref, agg_ref, w1_ref, b1_ref, al_ref, g_ref, be_ref,
                w2_ref, b2_ref, out_ref):
    h = x_ref[...] + agg_ref[0, :_N, :] + agg_ref[1, :_N, :]
    h = jnp.dot(h, w1_ref[...], preferred_element_type=jnp.float32) + b1_ref[...]
    mean = jnp.mean(h, axis=0, keepdims=True)
    cen = h - al_ref[...] * mean
    var = jnp.mean(cen * cen, axis=0, keepdims=True)
    h = g_ref[...] * cen / jnp.sqrt(var + 1e-5) + be_ref[...]
    h = jnp.maximum(h, 0.0)
    h = jnp.dot(h, w2_ref[...], preferred_element_type=jnp.float32) + b2_ref[...]
    out_ref[...] = jnp.maximum(h, 0.0)


def _dense_layer(x, agg, W1, b1, alpha, gamma, beta, W2, b2):
    return pl.pallas_call(
        _dense_body,
        out_shape=jax.ShapeDtypeStruct((_N, _D), jnp.float32),
    )(x, agg, W1, b1.reshape(1, _D), alpha.reshape(1, _D),
      gamma.reshape(1, _D), beta.reshape(1, _D), W2, b2.reshape(1, _D))


def kernel(x, edge_index, W1_0, b1_0, alpha_0, gamma_0, beta_0, W2_0, b2_0,
           W1_1, b1_1, alpha_1, gamma_1, beta_1, W2_1, b2_1):
    src1 = edge_index[0]
    dst1 = edge_index[1]
    agg0 = _segment_sum_sc(x, src1, dst1)
    h = _dense_layer(x, agg0, W1_0, b1_0, alpha_0, gamma_0, beta_0, W2_0, b2_0)
    agg1 = _segment_sum_sc(h, src1, dst1)
    h = _dense_layer(h, agg1, W1_1, b1_1, alpha_1, gamma_1, beta_1, W2_1, b2_1)
    return h
